# Initial kernel scaffold; baseline (speedup 1.0000x reference)
import jax
import jax.numpy as jnp
from jax.experimental import pallas as pl


def kernel(x, W_in, b_in, joint_embed, ln_g, ln_b, W_val, b_val, W_g1, b_g1, W_g2, b_g2, hn_g, hn_b, W_cls, b_cls):
    raise NotImplementedError("write your pallas kernel here")


# speedup vs baseline: 3.6634x; 3.6634x over previous
import functools

import jax
import jax.numpy as jnp
import numpy as np
from jax.experimental import pallas as pl
from jax.experimental.pallas import tpu as pltpu

_HAND_CONNECTIONS = [
    (0, 1), (1, 2), (2, 3), (3, 4),
    (0, 5), (5, 6), (6, 7), (7, 8),
    (5, 9), (9, 10), (10, 11), (11, 12),
    (9, 13), (13, 14), (14, 15), (15, 16),
    (13, 17), (17, 18), (18, 19), (19, 20),
    (0, 17),
]


def _build_edges(num_landmarks=21):
    edges = []
    for s, d in _HAND_CONNECTIONS:
        edges.append((s, d))
        edges.append((d, s))
    for j in range(num_landmarks):
        edges.append((j, j))
    return edges


_EDGES = _build_edges()
_D = 192
_GH = 96
_LAYERS = 3
_N = 21
_TB = 256


def _ln(v, g, b):
    mu = jnp.mean(v, axis=1, keepdims=True)
    var = jnp.mean((v - mu) ** 2, axis=1, keepdims=True)
    return (v - mu) * jax.lax.rsqrt(var + 1e-5) * g + b


def _body(x_ref, wbig_ref, bval_ref, bg1_ref, wg2_ref, bg2_ref, win_ref,
          bin_ref, je_ref, lng_ref, lnb_ref, hng_ref, hnb_ref, wcls_ref,
          bcls_ref, out_ref):
    tb = x_ref.shape[0]
    x = x_ref[...]
    win = win_ref[...]
    b_in = bin_ref[...]
    je = je_ref[...]

    rows = []
    for l in range(_N):
        hl = (x[:, 3 * l + 0:3 * l + 1] * win[0:1, :]
              + x[:, 3 * l + 1:3 * l + 2] * win[1:2, :]
              + x[:, 3 * l + 2:3 * l + 3] * win[2:3, :])
        rows.append(hl + b_in + je[l:l + 1, :])
    h = jnp.concatenate(rows, axis=0)

    wbig = wbig_ref[...]
    bval = bval_ref[...]
    bg1 = bg1_ref[...]
    wg2 = wg2_ref[...]
    bg2 = bg2_ref[...]
    lng = lng_ref[...]
    lnb = lnb_ref[...]

    for i in range(_LAYERS):
        xn = _ln(h, lng[i:i + 1, :], lnb[i:i + 1, :])
        vg = jnp.dot(xn, wbig[i], preferred_element_type=jnp.float32)
        v = vg[:, :_D] + bval[i:i + 1, :]
        a1 = vg[:, _D:_D + _GH] + bg1[i:i + 1, :]
        a2 = vg[:, _D + _GH:]
        w2 = wg2[i:i + 1, :]
        b2 = bg2[i, 0]
        agg = [None] * _N
        for (s, d) in _EDGES:
            sa = slice(s * tb, (s + 1) * tb)
            u = a1[sa] + a2[d * tb:(d + 1) * tb]
            gh = 0.5 * u * (1.0 + jax.lax.erf(u * 0.7071067811865476))
            gl = jnp.sum(gh * w2, axis=1, keepdims=True) + b2
            msg = jax.nn.sigmoid(gl) * v[sa]
            agg[d] = msg if agg[d] is None else agg[d] + msg
        h = h + jnp.concatenate(agg, axis=0)

    y = _ln(h, hng_ref[...], hnb_ref[...])
    pooled = y[0:tb]
    for l in range(1, _N):
        pooled = pooled + y[l * tb:(l + 1) * tb]
    pooled = pooled * (1.0 / _N)
    out_ref[...] = (jnp.dot(pooled, wcls_ref[...],
                            preferred_element_type=jnp.float32)
                    + bcls_ref[...])


@jax.jit
def kernel(x, W_in, b_in, joint_embed, ln_g, ln_b, W_val, b_val, W_g1, b_g1,
           W_g2, b_g2, hn_g, hn_b, W_cls, b_cls):
    batch = x.shape[0]
    xf = x.reshape(batch, _N * 3)
    wbig = jnp.concatenate(
        [W_val, W_g1[:, :_D, :], W_g1[:, _D:, :]], axis=2)
    wg2 = W_g2[:, :, 0]
    num_classes = W_cls.shape[1]

    grid = (batch // _TB,)

    def fixed(shape):
        nd = len(shape)
        return pl.BlockSpec(shape, lambda i, _nd=nd: (0,) * _nd)

    return pl.pallas_call(
        _body,
        grid=grid,
        in_specs=[
            pl.BlockSpec((_TB, _N * 3), lambda i: (i, 0)),
            fixed(wbig.shape),
            fixed((_LAYERS, _D)),
            fixed((_LAYERS, _GH)),
            fixed((_LAYERS, _GH)),
            fixed((_LAYERS, 1)),
            fixed((3, _D)),
            fixed((1, _D)),
            fixed((_N, _D)),
            fixed((_LAYERS, _D)),
            fixed((_LAYERS, _D)),
            fixed((1, _D)),
            fixed((1, _D)),
            fixed((_D, num_classes)),
            fixed((1, num_classes)),
        ],
        out_specs=pl.BlockSpec((_TB, num_classes), lambda i: (i, 0)),
        out_shape=jax.ShapeDtypeStruct((batch, num_classes), jnp.float32),
        compiler_params=pltpu.CompilerParams(
            dimension_semantics=("arbitrary",)),
    )(xf, wbig, b_val, b_g1, wg2, b_g2, W_in, b_in.reshape(1, _D),
      joint_embed, ln_g, ln_b, hn_g.reshape(1, _D), hn_b.reshape(1, _D),
      W_cls, b_cls.reshape(1, num_classes))
